# Initial kernel scaffold; baseline (speedup 1.0000x reference)
import jax
import jax.numpy as jnp
from jax.experimental import pallas as pl


def kernel(x, last_update, edge_index, t, msg, Wq, bq, Wk, bk, Wv, bv, We, Ws, bs, Wt, bt):
    raise NotImplementedError("write your pallas kernel here")


# speedup vs baseline: 3.4523x; 3.4523x over previous
import functools

import jax
import jax.numpy as jnp
from jax import lax
from jax.experimental import pallas as pl
from jax.experimental.pallas import tpu as pltpu
from jax.experimental.pallas import tpu_sc as plsc

N = 10000
E = 320000
D = 128
MSG = 16
TDIM = 100
H = 2
DH = 64
HD = H * DH

KV = 2 * HD
DN = 1280
NP = 10240

NW = 32
BPW = E // NW
CH = 80
NCH = BPW // CH
NG = CH // 16

NB = 1000
EB = 1000


def _proj_body(x_ref, wq, bq, wk, bk, wv, bv, ws, bs, q_o, k_o, v_o, s_o):
    xb = x_ref[...]
    q_o[...] = jnp.dot(xb, wq[...], preferred_element_type=jnp.float32) + bq[...]
    k_o[...] = jnp.dot(xb, wk[...], preferred_element_type=jnp.float32) + bk[...]
    v_o[...] = jnp.dot(xb, wv[...], preferred_element_type=jnp.float32) + bv[...]
    s_o[...] = jnp.dot(xb, ws[...], preferred_element_type=jnp.float32) + bs[...]


def _proj(x, wqT, bq, wkT, bk, wvT, bv, wsT, bs):
    w_spec = pl.BlockSpec((D, HD), lambda i: (0, 0))
    b_spec = pl.BlockSpec((1, HD), lambda i: (0, 0))
    out = jax.ShapeDtypeStruct((N, HD), jnp.float32)
    return pl.pallas_call(
        _proj_body,
        grid=(N // NB,),
        in_specs=[pl.BlockSpec((NB, D), lambda i: (i, 0)),
                  w_spec, b_spec, w_spec, b_spec, w_spec, b_spec, w_spec, b_spec],
        out_specs=[pl.BlockSpec((NB, HD), lambda i: (i, 0))] * 4,
        out_shape=[out, out, out, out],
    )(x, wqT, bq, wkT, bk, wvT, bv, wsT, bs)


def _sc_gather(kv, q_all, last_update, src, dst, t):
    mesh = plsc.VectorSubcoreMesh(core_axis_name="c", subcore_axis_name="s")

    @functools.partial(
        pl.kernel,
        mesh=mesh,
        out_type=[jax.ShapeDtypeStruct((E, KV), jnp.float32),
                  jax.ShapeDtypeStruct((E, HD), jnp.float32),
                  jax.ShapeDtypeStruct((E,), jnp.float32)],
        scratch_types=[pltpu.VMEM((CH,), jnp.int32),
                       pltpu.VMEM((CH,), jnp.int32),
                       pltpu.VMEM((CH,), jnp.float32),
                       pltpu.VMEM((CH,), jnp.float32),
                       pltpu.VMEM((CH, KV), jnp.float32),
                       pltpu.VMEM((CH, HD), jnp.float32),
                       pltpu.VMEM((N,), jnp.float32),
                       pltpu.SemaphoreType.DMA,
                       pltpu.SemaphoreType.DMA],
        compiler_params=pltpu.CompilerParams(needs_layout_passes=False),
    )
    def gather_kernel(kv_hbm, q_hbm, lu_hbm, src_hbm, dst_hbm, t_hbm,
                      kv_o, q_o, rel_o,
                      isrc, idst, tbuf, relbuf, kvbuf, qbuf, lubuf, sem1, sem2):
        wid = lax.axis_index("s") * 2 + lax.axis_index("c")
        base = wid * BPW
        pltpu.sync_copy(lu_hbm, lubuf)

        @pl.loop(0, NCH)
        def _(c):
            off = base + c * CH
            pltpu.sync_copy(src_hbm.at[pl.ds(off, CH)], isrc)
            pltpu.sync_copy(dst_hbm.at[pl.ds(off, CH)], idst)
            pltpu.sync_copy(t_hbm.at[pl.ds(off, CH)], tbuf)
            cp1 = pltpu.async_copy(kv_hbm.at[isrc], kvbuf, sem1)
            cp2 = pltpu.async_copy(q_hbm.at[idst], qbuf, sem2)
            for g in range(NG):
                sl = pl.ds(g * 16, 16)
                luv = plsc.load_gather(lubuf, [isrc[sl]])
                relbuf[sl] = luv - tbuf[sl]
            cp1.wait()
            cp2.wait()
            pltpu.sync_copy(kvbuf, kv_o.at[pl.ds(off, CH)])
            pltpu.sync_copy(qbuf, q_o.at[pl.ds(off, CH)])
            pltpu.sync_copy(relbuf, rel_o.at[pl.ds(off, CH)])

    return gather_kernel(kv, q_all, last_update, src, dst, t)


def _edge_body(kv_ref, q_ref, rel_ref, msg_ref, dst_ref, wet, wem, wt, bt_r,
               msg_o, den_o):
    k = kv_ref[:, 0:HD]
    v = kv_ref[:, HD:2 * HD]
    rel = rel_ref[...]
    enc = jnp.cos(rel * wt[...] + bt_r[...])
    e = (jnp.dot(enc, wet[...], preferred_element_type=jnp.float32)
         + jnp.dot(msg_ref[...], wem[...], preferred_element_type=jnp.float32))
    af = q_ref[...] * (k + e)
    a0 = jnp.sum(af[:, :DH], axis=1, keepdims=True) * (1.0 / 8.0)
    a1 = jnp.sum(af[:, DH:], axis=1, keepdims=True) * (1.0 / 8.0)
    ea0 = jnp.exp(a0)
    ea1 = jnp.exp(a1)
    lane = lax.broadcasted_iota(jnp.int32, (EB, HD), 1)
    w = jnp.where(lane < DH, ea0, ea1)
    msg_o[...] = w * (v + e)
    tgt = (dst_ref[...] & 7) * 16
    zero = jnp.zeros((EB, HD), jnp.float32)
    den_o[...] = (jnp.where(lane == tgt, ea0, zero)
                  + jnp.where(lane == tgt + 1, ea1, zero))


def _edge(kv_g, q_g, rel_col, msg, dst_col, wet, wem, wt, bt_r):
    eout = jax.ShapeDtypeStruct((E, HD), jnp.float32)
    return pl.pallas_call(
        _edge_body,
        grid=(E // EB,),
        in_specs=[pl.BlockSpec((EB, KV), lambda i: (i, 0)),
                  pl.BlockSpec((EB, HD), lambda i: (i, 0)),
                  pl.BlockSpec((EB, 1), lambda i: (i, 0)),
                  pl.BlockSpec((EB, MSG), lambda i: (i, 0)),
                  pl.BlockSpec((EB, 1), lambda i: (i, 0)),
                  pl.BlockSpec((TDIM, HD), lambda i: (0, 0)),
                  pl.BlockSpec((MSG, HD), lambda i: (0, 0)),
                  pl.BlockSpec((1, TDIM), lambda i: (0, 0)),
                  pl.BlockSpec((1, TDIM), lambda i: (0, 0))],
        out_specs=[pl.BlockSpec((EB, HD), lambda i: (i, 0))] * 2,
        out_shape=[eout, eout],
    )(kv_g, q_g, rel_col, msg, dst_col, wet, wem, wt, bt_r)


def _sc_scatter(contrib, spread, dst, zeros):
    mesh = plsc.VectorSubcoreMesh(core_axis_name="c", subcore_axis_name="s")
    rpt = NP // 16
    drpt = DN // 16

    @functools.partial(
        pl.kernel,
        mesh=mesh,
        out_type=[jax.ShapeDtypeStruct((2, NP, HD), jnp.float32),
                  jax.ShapeDtypeStruct((2, DN, HD), jnp.float32)],
        scratch_types=[pltpu.VMEM((CH,), jnp.int32),
                       pltpu.VMEM((CH,), jnp.int32),
                       pltpu.VMEM((CH, HD), jnp.float32),
                       pltpu.VMEM((CH, HD), jnp.float32),
                       pltpu.VMEM_SHARED((NP, HD), jnp.float32),
                       pltpu.VMEM_SHARED((DN, HD), jnp.float32)],
    )
    def scatter_kernel(contrib_hbm, spread_hbm, dst_hbm, zeros_hbm,
                       part_o, dpart_o, idx, idx8, rows, rows2, acc, dacc):
        cid = lax.axis_index("c")
        sid = lax.axis_index("s")
        wid = sid * 2 + cid
        base = wid * BPW
        row0 = sid * rpt
        drow0 = sid * drpt
        pltpu.sync_copy(zeros_hbm.at[pl.ds(row0, rpt)], acc.at[pl.ds(row0, rpt)])
        pltpu.sync_copy(zeros_hbm.at[pl.ds(drow0, drpt)], dacc.at[pl.ds(drow0, drpt)])
        plsc.subcore_barrier()

        @pl.loop(0, NCH)
        def _(c):
            off = base + c * CH
            pltpu.sync_copy(dst_hbm.at[pl.ds(off, CH)], idx)
            pltpu.sync_copy(contrib_hbm.at[pl.ds(off, CH)], rows)
            pltpu.sync_copy(spread_hbm.at[pl.ds(off, CH)], rows2)
            for g in range(NG):
                sl = pl.ds(g * 16, 16)
                idx8[sl] = lax.shift_right_logical(idx[sl], 3)
            pltpu.sync_copy(rows, acc.at[idx], add=True)
            pltpu.sync_copy(rows2, dacc.at[idx8], add=True)

        plsc.subcore_barrier()
        pltpu.sync_copy(acc.at[pl.ds(row0, rpt)], part_o.at[cid, pl.ds(row0, rpt)])
        pltpu.sync_copy(dacc.at[pl.ds(drow0, drpt)],
                        dpart_o.at[cid, pl.ds(drow0, drpt)])

    return scatter_kernel(contrib, spread, dst, zeros)


def _final_body(p_ref, den_ref, skip_ref, out_ref):
    num = p_ref[0, :, :] + p_ref[1, :, :]
    d0 = den_ref[:, 0:1]
    d1 = den_ref[:, 1:2]
    lane = lax.broadcasted_iota(jnp.int32, (NB, HD), 1)
    den = jnp.where(lane < DH, d0, d1) + 1e-16
    out_ref[...] = num / den + skip_ref[...]


def _final(parts, den, skip):
    return pl.pallas_call(
        _final_body,
        grid=(N // NB,),
        in_specs=[pl.BlockSpec((2, NB, HD), lambda i: (0, i, 0)),
                  pl.BlockSpec((NB, 2), lambda i: (i, 0)),
                  pl.BlockSpec((NB, HD), lambda i: (i, 0))],
        out_specs=pl.BlockSpec((NB, HD), lambda i: (i, 0)),
        out_shape=jax.ShapeDtypeStruct((N, HD), jnp.float32),
    )(parts, den, skip)


def kernel(x, last_update, edge_index, t, msg, Wq, bq, Wk, bk, Wv, bv, We, Ws, bs, Wt, bt):
    q_all, k_all, v_all, skip = _proj(
        x, Wq.T, bq[None, :], Wk.T, bk[None, :], Wv.T, bv[None, :], Ws.T, bs[None, :])
    kv = jnp.concatenate([k_all, v_all], axis=1)
    src = edge_index[0]
    dst = edge_index[1]
    kv_g, q_g, rel_g = _sc_gather(kv, q_all, last_update, src, dst, t)
    WeT = We.T
    contrib, spread = _edge(kv_g, q_g, rel_g[:, None], msg, dst[:, None],
                            WeT[:TDIM], WeT[TDIM:], Wt[:, 0][None, :], bt[None, :])
    parts, dparts = _sc_scatter(contrib, spread, dst,
                                jnp.zeros((NP, HD), jnp.float32))
    parts = parts[:, :N]
    den = (dparts[0, :N // 8] + dparts[1, :N // 8]).reshape(N, 16)[:, :2]
    return _final(parts, den, skip)
